# Initial kernel scaffold; baseline (speedup 1.0000x reference)
import jax
import jax.numpy as jnp
from jax.experimental import pallas as pl


def kernel(x, pe):
    raise NotImplementedError("write your pallas kernel here")


# speedup vs baseline: 1.4145x; 1.4145x over previous
import functools

import jax
import jax.numpy as jnp
from jax import lax
from jax.experimental import pallas as pl
from jax.experimental.pallas import tpu as pltpu
from jax.experimental.pallas import tpu_sc as plsc

_NC = 2
_NS = 16
_NW = _NC * _NS
_L = 16

_ROWS_CHUNK = 512


def _index_body(x_ref, idx_ref, *, scale_max):
    v = x_ref[...]
    lane = lax.broadcasted_iota(jnp.int32, (1, v.shape[1]), 1)
    even = (lane % 2) == 0
    colmin = jnp.min(v, axis=0, keepdims=True)
    colmax = jnp.max(v, axis=0, keepdims=True)
    mn0 = jnp.min(jnp.where(even, colmin, jnp.inf))
    mn1 = jnp.min(jnp.where(even, jnp.inf, colmin))
    mx0 = jnp.max(jnp.where(even, colmax, -jnp.inf))
    mx1 = jnp.max(jnp.where(even, -jnp.inf, colmax))
    mnv = jnp.where(even, mn0, mn1)
    dnv = jnp.where(even, mx0 - mn0, mx1 - mn1) + 1e-8
    xn = jnp.clip((v - mnv) / dnv, 0.0, 1.0)
    idx_ref[...] = (xn * scale_max).astype(jnp.int32)


def _compute_indices(xr, scale_max):
    return pl.pallas_call(
        functools.partial(_index_body, scale_max=scale_max),
        out_shape=jax.ShapeDtypeStruct(xr.shape, jnp.int32),
    )(xr)


def _make_sc_gather(max_len, d_half, n):
    rows_w = n // _NW
    n_chunks = rows_w // _ROWS_CHUNK
    assert rows_w % _ROWS_CHUNK == 0
    d_out = 2 * d_half
    mesh = plsc.VectorSubcoreMesh(core_axis_name="c", subcore_axis_name="s")

    @functools.partial(
        pl.kernel,
        mesh=mesh,
        out_type=jax.ShapeDtypeStruct((n * d_out,), jnp.float32),
        compiler_params=pltpu.CompilerParams(needs_layout_passes=False),
        scratch_types=[
            pltpu.VMEM((max_len * d_half,), jnp.float32),
            pltpu.VMEM((2 * _ROWS_CHUNK,), jnp.int32),
            pltpu.VMEM((_ROWS_CHUNK * d_out,), jnp.float32),
        ],
    )
    def sc_gather(pe_hbm, idx_hbm, out_hbm, pe_v, idx_v, out_v):
        wid = lax.axis_index("s") * _NC + lax.axis_index("c")
        row0 = wid * rows_w
        pltpu.sync_copy(pe_hbm, pe_v)
        iota = lax.iota(jnp.int32, _L)

        def chunk(ci, carry):
            rbase = row0 + ci * _ROWS_CHUNK
            pltpu.sync_copy(
                idx_hbm.at[pl.ds(2 * rbase, 2 * _ROWS_CHUNK)], idx_v
            )

            def group(g, c2):
                pos = (g * _L + iota) * 2
                ixb = plsc.load_gather(idx_v, [pos]) * d_half
                iyb = plsc.load_gather(idx_v, [pos + 1]) * d_half
                ob = pos * d_half
                for c in range(d_half):
                    vx = plsc.load_gather(pe_v, [ixb + c])
                    plsc.store_scatter(out_v, [ob + c], vx)
                    vy = plsc.load_gather(pe_v, [iyb + c])
                    plsc.store_scatter(out_v, [ob + (d_half + c)], vy)
                return c2

            lax.fori_loop(0, _ROWS_CHUNK // _L, group, 0)
            pltpu.sync_copy(
                out_v, out_hbm.at[pl.ds(rbase * d_out, _ROWS_CHUNK * d_out)]
            )
            return carry

        lax.fori_loop(0, n_chunks, chunk, 0)

    return sc_gather


def kernel(x, pe):
    n, two = x.shape
    max_len, d_half = pe.shape

    xr = x.reshape(n * two // 128, 128)
    idx2d = _compute_indices(xr, float(max_len - 1))
    idx_flat = idx2d.reshape(n * two)

    out_flat = _make_sc_gather(max_len, d_half, n)(pe.reshape(-1), idx_flat)
    return out_flat.reshape(n, 2 * d_half)


# speedup vs baseline: 1.7461x; 1.2345x over previous
import functools

import jax
import jax.numpy as jnp
from jax import lax
from jax.experimental import pallas as pl
from jax.experimental.pallas import tpu as pltpu
from jax.experimental.pallas import tpu_sc as plsc

_NC = 2
_NS = 16
_NW = _NC * _NS
_L = 16

_ROWS_CHUNK = 512


def _index_body(x_ref, idx_ref, *, scale_max):
    v = x_ref[...]
    lane = lax.broadcasted_iota(jnp.int32, (1, v.shape[1]), 1)
    even = (lane % 2) == 0
    colmin = jnp.min(v, axis=0, keepdims=True)
    colmax = jnp.max(v, axis=0, keepdims=True)
    mn0 = jnp.min(jnp.where(even, colmin, jnp.inf))
    mn1 = jnp.min(jnp.where(even, jnp.inf, colmin))
    mx0 = jnp.max(jnp.where(even, colmax, -jnp.inf))
    mx1 = jnp.max(jnp.where(even, -jnp.inf, colmax))
    mnv = jnp.where(even, mn0, mn1)
    dnv = jnp.where(even, mx0 - mn0, mx1 - mn1) + 1e-8
    xn = jnp.clip((v - mnv) / dnv, 0.0, 1.0)
    idx_ref[...] = (xn * scale_max).astype(jnp.int32)


def _compute_indices(xr, scale_max):
    return pl.pallas_call(
        functools.partial(_index_body, scale_max=scale_max),
        out_shape=jax.ShapeDtypeStruct(xr.shape, jnp.int32),
    )(xr)


def _make_sc_gather(max_len, d_half, n):
    rows_w = n // _NW
    n_chunks = rows_w // _ROWS_CHUNK
    assert rows_w % _ROWS_CHUNK == 0
    d_out = 2 * d_half
    mesh = plsc.VectorSubcoreMesh(core_axis_name="c", subcore_axis_name="s")

    @functools.partial(
        pl.kernel,
        mesh=mesh,
        out_type=jax.ShapeDtypeStruct((n * d_out,), jnp.float32),
        compiler_params=pltpu.CompilerParams(needs_layout_passes=False),
        scratch_types=[
            pltpu.VMEM((max_len * d_half,), jnp.float32),
            pltpu.VMEM((2 * _ROWS_CHUNK,), jnp.int32),
            pltpu.VMEM((_ROWS_CHUNK * d_out,), jnp.float32),
        ],
    )
    def sc_gather(pe_hbm, idx_hbm, out_hbm, pe_v, idx_v, out_v):
        wid = lax.axis_index("s") * _NC + lax.axis_index("c")
        row0 = wid * rows_w
        pltpu.sync_copy(pe_hbm, pe_v)
        iota = lax.iota(jnp.int32, _L)

        def chunk(ci, carry):
            rbase = row0 + ci * _ROWS_CHUNK
            pltpu.sync_copy(
                idx_hbm.at[pl.ds(2 * rbase, 2 * _ROWS_CHUNK)], idx_v
            )

            @plsc.parallel_loop(0, _ROWS_CHUNK // _L, unroll=2)
            def group(g):
                pos = (g * _L + iota) * 2
                ixb = plsc.load_gather(idx_v, [pos]) * d_half
                iyb = plsc.load_gather(idx_v, [pos + 1]) * d_half
                ob = pos * d_half
                for c in range(d_half):
                    vx = plsc.load_gather(pe_v, [ixb + c])
                    plsc.store_scatter(out_v, [ob + c], vx)
                    vy = plsc.load_gather(pe_v, [iyb + c])
                    plsc.store_scatter(out_v, [ob + (d_half + c)], vy)
            pltpu.sync_copy(
                out_v, out_hbm.at[pl.ds(rbase * d_out, _ROWS_CHUNK * d_out)]
            )
            return carry

        lax.fori_loop(0, n_chunks, chunk, 0)

    return sc_gather


def kernel(x, pe):
    n, two = x.shape
    max_len, d_half = pe.shape

    xr = x.reshape(n * two // 128, 128)
    idx2d = _compute_indices(xr, float(max_len - 1))
    idx_flat = idx2d.reshape(n * two)

    out_flat = _make_sc_gather(max_len, d_half, n)(pe.reshape(-1), idx_flat)
    return out_flat.reshape(n, 2 * d_half)


# speedup vs baseline: 7.2491x; 4.1516x over previous
import functools

import jax
import jax.numpy as jnp
from jax import lax
from jax.experimental import pallas as pl
from jax.experimental.pallas import tpu as pltpu
from jax.experimental.pallas import tpu_sc as plsc

_NC = 2
_NS = 16
_NW = _NC * _NS
_L = 16

_ROWS_CHUNK = 512


def _index_body(x_ref, idx_ref, *, scale_max):
    v = x_ref[...]
    lane = lax.broadcasted_iota(jnp.int32, (1, v.shape[1]), 1)
    even = (lane % 2) == 0
    colmin = jnp.min(v, axis=0, keepdims=True)
    colmax = jnp.max(v, axis=0, keepdims=True)
    mn0 = jnp.min(jnp.where(even, colmin, jnp.inf))
    mn1 = jnp.min(jnp.where(even, jnp.inf, colmin))
    mx0 = jnp.max(jnp.where(even, colmax, -jnp.inf))
    mx1 = jnp.max(jnp.where(even, -jnp.inf, colmax))
    mnv = jnp.where(even, mn0, mn1)
    dnv = jnp.where(even, mx0 - mn0, mx1 - mn1) + 1e-8
    xn = jnp.clip((v - mnv) / dnv, 0.0, 1.0)
    idx_ref[...] = (xn * scale_max).astype(jnp.int32)


def _compute_indices(xr, scale_max):
    return pl.pallas_call(
        functools.partial(_index_body, scale_max=scale_max),
        out_shape=jax.ShapeDtypeStruct(xr.shape, jnp.int32),
    )(xr)


def _make_sc_gather(max_len, d_half, n):
    rows_w = n // _NW
    n_chunks = rows_w // _ROWS_CHUNK
    assert rows_w % _ROWS_CHUNK == 0
    d_out = 2 * d_half
    mesh = plsc.VectorSubcoreMesh(core_axis_name="c", subcore_axis_name="s")

    @functools.partial(
        pl.kernel,
        mesh=mesh,
        out_type=jax.ShapeDtypeStruct((n * d_out,), jnp.float32),
        compiler_params=pltpu.CompilerParams(needs_layout_passes=False),
        scratch_types=[
            pltpu.VMEM((max_len * d_half,), jnp.float32),
            pltpu.VMEM((2 * _ROWS_CHUNK,), jnp.int32),
            pltpu.VMEM((_ROWS_CHUNK * d_out,), jnp.float32),
        ],
    )
    def sc_gather(pe_hbm, idx_hbm, out_hbm, pe_v, idx_v, out_v):
        wid = lax.axis_index("s") * _NC + lax.axis_index("c")
        row0 = wid * rows_w
        pltpu.sync_copy(pe_hbm, pe_v)
        iota = lax.iota(jnp.int32, _L)

        def chunk(ci, carry):
            rbase = row0 + ci * _ROWS_CHUNK
            pltpu.sync_copy(
                idx_hbm.at[pl.ds(2 * rbase, 2 * _ROWS_CHUNK)], idx_v
            )

            @plsc.parallel_loop(0, _ROWS_CHUNK // 8, unroll=2)
            def group(g):
                iv = idx_v[pl.ds(g * _L, _L)] * d_half
                for k in range(8):
                    ix = iv[2 * k]
                    iy = iv[2 * k + 1]
                    ob = (g * 8 + k) * d_out
                    for c in range(0, d_half, _L):
                        out_v[pl.ds(ob + c, _L)] = pe_v[pl.ds(ix + c, _L)]
                    for c in range(0, d_half, _L):
                        out_v[pl.ds(ob + d_half + c, _L)] = pe_v[
                            pl.ds(iy + c, _L)
                        ]
            pltpu.sync_copy(
                out_v, out_hbm.at[pl.ds(rbase * d_out, _ROWS_CHUNK * d_out)]
            )
            return carry

        lax.fori_loop(0, n_chunks, chunk, 0)

    return sc_gather


def kernel(x, pe):
    n, two = x.shape
    max_len, d_half = pe.shape

    xr = x.reshape(n * two // 128, 128)
    idx2d = _compute_indices(xr, float(max_len - 1))
    idx_flat = idx2d.reshape(n * two)

    out_flat = _make_sc_gather(max_len, d_half, n)(pe.reshape(-1), idx_flat)
    return out_flat.reshape(n, 2 * d_half)


# speedup vs baseline: 7.2961x; 1.0065x over previous
import functools

import jax
import jax.numpy as jnp
from jax import lax
from jax.experimental import pallas as pl
from jax.experimental.pallas import tpu as pltpu
from jax.experimental.pallas import tpu_sc as plsc

_NC = 2
_NS = 16
_NW = _NC * _NS
_L = 16

_CH = 128


def _make_fused(max_len, d_half, n):
    rows_w = n // _NW
    n_pairs = rows_w // (2 * _CH)
    assert rows_w % (2 * _CH) == 0
    d_out = 2 * d_half
    nxv = 2 * rows_w // _L
    scale_max = float(max_len - 1)
    mesh = plsc.VectorSubcoreMesh(core_axis_name="c", subcore_axis_name="s")

    @functools.partial(
        pl.kernel,
        mesh=mesh,
        out_type=jax.ShapeDtypeStruct((n * d_out,), jnp.float32),
        compiler_params=pltpu.CompilerParams(needs_layout_passes=False),
        scratch_types=[
            pltpu.VMEM((max_len * d_half,), jnp.float32),
            pltpu.VMEM((2 * rows_w,), jnp.float32),
            pltpu.VMEM((_CH * d_out,), jnp.float32),
            pltpu.VMEM((_CH * d_out,), jnp.float32),
            pltpu.VMEM((2 * _CH,), jnp.int32),
            pltpu.VMEM((2 * _L,), jnp.float32),
            pltpu.VMEM((_NW * 2 * _L,), jnp.float32),
            pltpu.VMEM_SHARED((_NW * 2 * _L,), jnp.float32),
            pltpu.SemaphoreType.DMA,
            pltpu.SemaphoreType.DMA,
            pltpu.SemaphoreType.DMA,
            pltpu.SemaphoreType.DMA,
        ],
    )
    def fused(
        x_hbm,
        pe_hbm,
        out_hbm,
        pe_v,
        x_v,
        out_v0,
        out_v1,
        idx_s,
        red_v,
        all_v,
        shared,
        sem_pe,
        sem_x,
        sem_o0,
        sem_o1,
    ):
        wid = lax.axis_index("s") * _NC + lax.axis_index("c")
        row0 = wid * rows_w
        pe_cp = pltpu.async_copy(pe_hbm, pe_v, sem_pe)
        pltpu.async_copy(x_hbm.at[pl.ds(2 * row0, 2 * rows_w)], x_v, sem_x).wait()

        iota = lax.iota(jnp.int32, _L)
        even = (iota % 2) == 0
        inf = jnp.float32(jnp.inf)

        def red(i, mnmx):
            mn, mx = mnmx
            v = x_v[pl.ds(i * _L, _L)]
            return jnp.minimum(mn, v), jnp.maximum(mx, v)

        mn, mx = lax.fori_loop(
            0, nxv, red, (jnp.full((_L,), inf), jnp.full((_L,), -inf))
        )
        red_v[pl.ds(0, _L)] = mn
        red_v[pl.ds(_L, _L)] = mx
        pltpu.sync_copy(red_v, shared.at[pl.ds(wid * 2 * _L, 2 * _L)])
        plsc.subcore_barrier()
        pltpu.sync_copy(shared, all_v)

        def red2(w, mnmx):
            mn, mx = mnmx
            a = all_v[pl.ds(w * 2 * _L, _L)]
            b = all_v[pl.ds(w * 2 * _L + _L, _L)]
            return jnp.minimum(mn, a), jnp.maximum(mx, b)

        mn, mx = lax.fori_loop(
            0, _NW, red2, (jnp.full((_L,), inf), jnp.full((_L,), -inf))
        )
        mnx = jnp.min(jnp.where(even, mn, inf))
        mny = jnp.min(jnp.where(even, inf, mn))
        mxx = jnp.max(jnp.where(even, mx, -inf))
        mxy = jnp.max(jnp.where(even, -inf, mx))
        mnv = jnp.where(even, mnx, mny)
        dnv = jnp.where(even, mxx - mnx, mxy - mny) + 1e-8

        def do_chunk(c, out_v, sem):
            def mkidx(g, carry):
                v = x_v[pl.ds((2 * c * _CH) + g * _L, _L)]
                xn = jnp.clip((v - mnv) / dnv, 0.0, 1.0)
                idx_s[pl.ds(g * _L, _L)] = (xn * scale_max).astype(
                    jnp.int32
                ) * d_half
                return carry

            lax.fori_loop(0, 2 * _CH // _L, mkidx, 0)

            @pl.when(c >= 2)
            def _():
                pltpu.make_async_copy(
                    out_v,
                    out_hbm.at[pl.ds((row0 + c * _CH) * d_out, _CH * d_out)],
                    sem,
                ).wait()

            @plsc.parallel_loop(0, 2 * _CH // _L, unroll=2)
            def group(g):
                iv = idx_s[pl.ds(g * _L, _L)]
                for k in range(8):
                    ix = iv[2 * k]
                    iy = iv[2 * k + 1]
                    ob = (g * 8 + k) * d_out
                    for cc in range(0, d_half, _L):
                        out_v[pl.ds(ob + cc, _L)] = pe_v[pl.ds(ix + cc, _L)]
                    for cc in range(0, d_half, _L):
                        out_v[pl.ds(ob + d_half + cc, _L)] = pe_v[
                            pl.ds(iy + cc, _L)
                        ]

            pltpu.async_copy(
                out_v,
                out_hbm.at[pl.ds((row0 + c * _CH) * d_out, _CH * d_out)],
                sem,
            )

        pe_cp.wait()

        def pair(ci, carry):
            do_chunk(2 * ci, out_v0, sem_o0)
            do_chunk(2 * ci + 1, out_v1, sem_o1)
            return carry

        lax.fori_loop(0, n_pairs, pair, 0)

        last = 2 * n_pairs - 1
        pltpu.make_async_copy(
            out_v0,
            out_hbm.at[pl.ds((row0 + (last - 1) * _CH) * d_out, _CH * d_out)],
            sem_o0,
        ).wait()
        pltpu.make_async_copy(
            out_v1,
            out_hbm.at[pl.ds((row0 + last * _CH) * d_out, _CH * d_out)],
            sem_o1,
        ).wait()

    return fused


def kernel(x, pe):
    n, two = x.shape
    max_len, d_half = pe.shape
    out_flat = _make_fused(max_len, d_half, n)(x.reshape(-1), pe.reshape(-1))
    return out_flat.reshape(n, 2 * d_half)
